# baseline (speedup 1.0000x reference)
import functools

import jax
import jax.numpy as jnp
from jax import lax
from jax.experimental import pallas as pl
from jax.experimental.pallas import tpu as pltpu
from jax.experimental.pallas import tpu_sc as plsc

NV = 6890
P = 1024
NVP = 6912
NQ = NVP // 4
VW = 128
KT = 2304
NKT = NVP // KT
WRB = 48
NWRB = NQ // WRB


def _repack(gm, v):
    def rp_fn(gm_ref, v_ref, gmq_ref, vp_ref):
        x = gm_ref[...]
        xp = jnp.pad(x, ((0, 0), (0, NVP - NV)))
        gmq_ref[...] = pltpu.bitcast(xp, jnp.int32)

        @pl.when(pl.program_id(0) == 0)
        def _():
            vp_ref[...] = jnp.pad(v_ref[...], ((0, 0), (0, VW - 3)))

    return pl.pallas_call(
        rp_fn,
        grid=(NWRB,),
        in_specs=[
            pl.BlockSpec((4 * WRB, NV), lambda r: (r, 0)),
            pl.BlockSpec((NV, 3), lambda r: (0, 0)),
        ],
        out_specs=[
            pl.BlockSpec((WRB, NVP), lambda r: (r, 0)),
            pl.BlockSpec((NV, VW), lambda r: (0, 0)),
        ],
        out_shape=[
            jax.ShapeDtypeStruct((NQ, NVP), jnp.int32),
            jax.ShapeDtypeStruct((NV, VW), jnp.float32),
        ],
    )(gm, v)


def _sc_gather(pc, pcq, gmq, vpad):
    info = plsc.get_sparse_core_info()
    nw = info.num_cores * info.num_subcores
    rpw = P // nw
    qtr = rpw // 4
    nch = 4

    mesh = plsc.VectorSubcoreMesh(core_axis_name="c", subcore_axis_name="s")

    @functools.partial(
        pl.kernel,
        mesh=mesh,
        out_type=[
            jax.ShapeDtypeStruct((P, NVP), jnp.int32),
            jax.ShapeDtypeStruct((P, VW), jnp.float32),
        ],
        scratch_types=[
            pltpu.VMEM((rpw,), jnp.int32),
            pltpu.VMEM((nch, qtr), jnp.int32),
            pltpu.VMEM((2, qtr, NVP), jnp.int32),
            pltpu.VMEM((rpw, VW), jnp.float32),
            pltpu.SemaphoreType.DMA,
            pltpu.SemaphoreType.DMA,
        ],
    )
    def sc_fn(pc_hbm, pcq_hbm, gmq_hbm, vpad_hbm, grow_hbm, vpg_hbm,
              vidx_v, idx_v, rows_v, vrows_v, vsem, sem):
        wid = lax.axis_index("s") * info.num_cores + lax.axis_index("c")
        base = wid * rpw
        pltpu.sync_copy(pc_hbm.at[pl.ds(base, rpw)], vidx_v)
        cpv = pltpu.async_copy(vpad_hbm.at[vidx_v], vrows_v, vsem)
        for ch in range(nch):
            pltpu.sync_copy(pcq_hbm.at[pl.ds(base + ch * qtr, qtr)],
                            idx_v.at[ch])
        cps = [None] * nch
        for ch in range(2):
            cps[ch] = pltpu.async_copy(gmq_hbm.at[idx_v.at[ch]],
                                       rows_v.at[ch % 2], sem)
        for ch in range(nch):
            cps[ch].wait()
            pltpu.sync_copy(rows_v.at[ch % 2],
                            grow_hbm.at[pl.ds(base + ch * qtr, qtr)])
            if ch + 2 < nch:
                cps[ch + 2] = pltpu.async_copy(gmq_hbm.at[idx_v.at[ch + 2]],
                                               rows_v.at[ch % 2], sem)
        cpv.wait()
        pltpu.sync_copy(vrows_v, vpg_hbm.at[pl.ds(base, rpw)])

    return sc_fn(pc, pcq, gmq, vpad)


def _tc_loss(pc_row, psh_col, vpg, grow):
    def tc_fn(pc_ref, psh_ref, vp_ref, g_ref, out_ref, acc_ref):
        kt = pl.program_id(0)

        @pl.when(kt == 0)
        def _():
            acc_ref[...] = jnp.zeros((P, P), jnp.float32)

        w = g_ref[...]
        ext = (w >> psh_ref[...]) & 1
        kio = lax.broadcasted_iota(jnp.int32, (KT, P), 0) + kt * KT
        oh = (kio == pc_ref[...]).astype(jnp.bfloat16)
        acc_ref[...] += jnp.dot(ext.astype(jnp.bfloat16), oh,
                                preferred_element_type=jnp.float32)

        @pl.when(kt == NKT - 1)
        def _():
            vp = vp_ref[...]
            s = jnp.zeros((P, P), jnp.float32)
            for c in range(3):
                col = vp[:, c:c + 1]
                e = (lax.broadcasted_iota(jnp.int32, (1, VW), 1) == c
                     ).astype(jnp.float32)
                row = lax.dot_general(e, vp, (((1,), (1,)), ((), ())),
                                      preferred_element_type=jnp.float32)
                d = col - row
                s = s + d * d
            big = jnp.float32(3.0e37)
            sm = jnp.where(acc_ref[...] > 0.5, s, big)
            rmin = jnp.min(sm, axis=1, keepdims=True)
            rmin = jnp.where(rmin >= big * 0.5, s[:, 0:1], rmin)
            out_ref[0, 0] = jnp.mean(jnp.tanh(jnp.sqrt(rmin)))

    out = pl.pallas_call(
        tc_fn,
        grid=(NKT,),
        in_specs=[
            pl.BlockSpec((1, P), lambda kt: (0, 0)),
            pl.BlockSpec((P, 1), lambda kt: (0, 0)),
            pl.BlockSpec((P, VW), lambda kt: (0, 0)),
            pl.BlockSpec((P, KT), lambda kt: (0, kt)),
        ],
        out_specs=pl.BlockSpec(memory_space=pltpu.SMEM),
        out_shape=jax.ShapeDtypeStruct((1, 1), jnp.float32),
        scratch_shapes=[pltpu.VMEM((P, P), jnp.float32)],
    )(pc_row, psh_col, vpg, grow)
    return out[0, 0]


def kernel(presented_contact, vertices, geomask):
    pc = presented_contact.astype(jnp.int32)
    gm8 = geomask.view(jnp.uint8)
    gmq, vpad = _repack(gm8, vertices[0])
    grow, vpg = _sc_gather(pc, pc >> 2, gmq, vpad)
    psh = ((pc & 3) * 8).reshape(P, 1)
    return _tc_loss(pc.reshape(1, P), psh, vpg, grow)
